# baseline (speedup 1.0000x reference)
import jax
import jax.numpy as jnp
from jax.experimental import pallas as pl
from jax.experimental.pallas import tpu as pltpu

N = 1024
IN_CH = 1024
HID = 64
OUT_CH = 64

_VMEM = pl.BlockSpec(memory_space=pltpu.VMEM)


def _gcn_body(x_ref, W1_ref, Wc1t_ref, b1_ref, bc1_ref, Wc2_ref, bc2_ref,
              W2t_ref, h2_ref):
    del W2t_ref
    x = x_ref[...]
    xb = x.astype(jnp.bfloat16)
    ada = jnp.dot(xb, W1_ref[...].astype(jnp.bfloat16),
                  preferred_element_type=jnp.float32)
    m = jnp.where(ada + b1_ref[...] != 0.0, 1.0, 0.0)
    deg = jnp.sum(m, axis=0, keepdims=True).T + 1.0
    dinv = jax.lax.rsqrt(deg)

    xw = jax.lax.dot_general(x, Wc1t_ref[...], (((1,), (1,)), ((), ())),
                             preferred_element_type=jnp.float32)
    y = xw * dinv
    z = jax.lax.dot_general(m, y, (((0,), (0,)), ((), ())),
                            preferred_element_type=jnp.float32) + y
    h1 = jnp.maximum(z * dinv + bc1_ref[...], 0.0)

    xw2 = jnp.dot(h1, Wc2_ref[...], preferred_element_type=jnp.float32)
    y2 = xw2 * dinv
    z2 = jax.lax.dot_general(m, y2, (((0,), (0,)), ((), ())),
                             preferred_element_type=jnp.float32) + y2
    h2_ref[...] = z2 * dinv + bc2_ref[...]


def _readout_body(v_ref, W2t_ref, b2_ref, o_ref):
    o_ref[...] = jax.lax.dot_general(
        v_ref[...], W2t_ref[...], (((1,), (1,)), ((), ())),
        preferred_element_type=jnp.float32) + b2_ref[...]


def kernel(x, W1, b1, Wc1, bc1, Wc2, bc2, W2, b2):
    W2t = W2.T
    h2 = pl.pallas_call(
        _gcn_body,
        in_specs=[_VMEM] * 8,
        out_specs=_VMEM,
        out_shape=jax.ShapeDtypeStruct((N, OUT_CH), jnp.float32),
    )(x, W1, Wc1.T, b1.reshape(1, IN_CH), bc1.reshape(1, HID), Wc2,
      bc2.reshape(1, OUT_CH), W2t)

    v = h2.reshape(1, N * OUT_CH)
    out = pl.pallas_call(
        _readout_body,
        in_specs=[_VMEM] * 3,
        out_specs=_VMEM,
        out_shape=jax.ShapeDtypeStruct((1, OUT_CH), jnp.float32),
    )(v, W2t, b2.reshape(1, OUT_CH))
    return out.reshape(OUT_CH)


# speedup vs baseline: 1.2290x; 1.2290x over previous
import jax
import jax.numpy as jnp
from jax.experimental import pallas as pl
from jax.experimental.pallas import tpu as pltpu

N = 1024
IN_CH = 1024
HID = 64
OUT_CH = 64

_VMEM = pl.BlockSpec(memory_space=pltpu.VMEM)


def _gcn_body(x_ref, W1_ref, Wc1t_ref, b1_ref, bc1_ref, Wc2_ref, bc2_ref,
              h2_ref):
    x = x_ref[...]
    xb = x.astype(jnp.bfloat16)
    ada = jnp.dot(xb, W1_ref[...].astype(jnp.bfloat16),
                  preferred_element_type=jnp.float32)
    m32 = jnp.where(ada + b1_ref[...] != 0.0, 1.0, 0.0)
    m = m32.astype(jnp.bfloat16)
    deg = jnp.sum(m32, axis=0, keepdims=True).T + 1.0
    dinv = jax.lax.rsqrt(deg)

    xw = jax.lax.dot_general(x, Wc1t_ref[...], (((1,), (1,)), ((), ())),
                             preferred_element_type=jnp.float32)
    y = xw * dinv
    z = jax.lax.dot_general(m, y.astype(jnp.bfloat16),
                            (((0,), (0,)), ((), ())),
                            preferred_element_type=jnp.float32) + y
    h1 = jnp.maximum(z * dinv + bc1_ref[...], 0.0)

    xw2 = jnp.dot(h1, Wc2_ref[...], preferred_element_type=jnp.float32)
    y2 = xw2 * dinv
    z2 = jax.lax.dot_general(m, y2.astype(jnp.bfloat16),
                             (((0,), (0,)), ((), ())),
                             preferred_element_type=jnp.float32) + y2
    h2_ref[...] = (z2 * dinv + bc2_ref[...]).astype(jnp.bfloat16)


def _readout_body(v_ref, W2t_ref, b2_ref, o_ref):
    o_ref[...] = jax.lax.dot_general(
        v_ref[...].astype(jnp.float32), W2t_ref[...],
        (((1,), (1,)), ((), ())),
        preferred_element_type=jnp.float32) + b2_ref[...]


def kernel(x, W1, b1, Wc1, bc1, Wc2, bc2, W2, b2):
    W2t = W2.T
    h2 = pl.pallas_call(
        _gcn_body,
        in_specs=[_VMEM] * 7,
        out_specs=_VMEM,
        out_shape=jax.ShapeDtypeStruct((N, OUT_CH), jnp.bfloat16),
    )(x, W1, Wc1.T, b1.reshape(1, IN_CH), bc1.reshape(1, HID), Wc2,
      bc2.reshape(1, OUT_CH))

    v = h2.reshape(1, N * OUT_CH)
    out = pl.pallas_call(
        _readout_body,
        in_specs=[_VMEM] * 3,
        out_specs=_VMEM,
        out_shape=jax.ShapeDtypeStruct((1, OUT_CH), jnp.float32),
    )(v, W2t, b2.reshape(1, OUT_CH))
    return out.reshape(OUT_CH)


# speedup vs baseline: 1.2349x; 1.0047x over previous
import jax
import jax.numpy as jnp
from jax.experimental import pallas as pl
from jax.experimental.pallas import tpu as pltpu

N = 1024
IN_CH = 1024
HID = 64
OUT_CH = 64

_VMEM = pl.BlockSpec(memory_space=pltpu.VMEM)


def _gcn_body(x_ref, W1_ref, Wc1t_ref, b1_ref, bc1_ref, Wc2_ref, bc2_ref,
              h2_ref):
    x = x_ref[...]
    xb = x.astype(jnp.bfloat16)
    ada = jnp.dot(xb, W1_ref[...].astype(jnp.bfloat16),
                  preferred_element_type=jnp.float32)
    m32 = jnp.where(ada + b1_ref[...] != 0.0, 1.0, 0.0)
    m = m32.astype(jnp.bfloat16)
    deg = jnp.sum(m32, axis=0, keepdims=True).T + 1.0
    dinv = jax.lax.rsqrt(deg)

    xw = jax.lax.dot_general(x, Wc1t_ref[...], (((1,), (1,)), ((), ())),
                             preferred_element_type=jnp.float32)
    y = xw * dinv
    z = jax.lax.dot_general(m, y.astype(jnp.bfloat16),
                            (((0,), (0,)), ((), ())),
                            preferred_element_type=jnp.float32) + y
    h1 = jnp.maximum(z * dinv + bc1_ref[...], 0.0)

    xw2 = jnp.dot(h1, Wc2_ref[...], preferred_element_type=jnp.float32)
    y2 = xw2 * dinv
    z2 = jax.lax.dot_general(m, y2.astype(jnp.bfloat16),
                             (((0,), (0,)), ((), ())),
                             preferred_element_type=jnp.float32) + y2
    h2_ref[...] = (z2 * dinv + bc2_ref[...]).astype(jnp.bfloat16)


NCHUNK = 4
CHUNK = (N * OUT_CH) // NCHUNK


def _readout_body(v_ref, W2t_hbm, b2_ref, o_ref, buf, sem):
    cps = [pltpu.make_async_copy(W2t_hbm.at[:, pl.ds(i * CHUNK, CHUNK)],
                                 buf.at[i], sem.at[i])
           for i in range(NCHUNK)]
    for cp in cps:
        cp.start()
    v = v_ref[...].astype(jnp.float32)
    acc = b2_ref[...]
    for i in range(NCHUNK):
        cps[i].wait()
        acc = acc + jax.lax.dot_general(
            v[:, i * CHUNK:(i + 1) * CHUNK], buf[i],
            (((1,), (1,)), ((), ())), preferred_element_type=jnp.float32)
    o_ref[...] = acc


def kernel(x, W1, b1, Wc1, bc1, Wc2, bc2, W2, b2):
    W2t = W2.T
    h2 = pl.pallas_call(
        _gcn_body,
        in_specs=[_VMEM] * 7,
        out_specs=_VMEM,
        out_shape=jax.ShapeDtypeStruct((N, OUT_CH), jnp.bfloat16),
    )(x, W1, Wc1.T, b1.reshape(1, IN_CH), bc1.reshape(1, HID), Wc2,
      bc2.reshape(1, OUT_CH))

    v = h2.reshape(1, N * OUT_CH)
    out = pl.pallas_call(
        _readout_body,
        in_specs=[_VMEM,
                  pl.BlockSpec(memory_space=pltpu.MemorySpace.HBM),
                  _VMEM],
        out_specs=_VMEM,
        out_shape=jax.ShapeDtypeStruct((1, OUT_CH), jnp.float32),
        scratch_shapes=[pltpu.VMEM((NCHUNK, OUT_CH, CHUNK), jnp.float32),
                        pltpu.SemaphoreType.DMA((NCHUNK,))],
    )(v, W2t, b2.reshape(1, OUT_CH))
    return out.reshape(OUT_CH)
